# Initial kernel scaffold; baseline (speedup 1.0000x reference)
import jax
import jax.numpy as jnp
from jax.experimental import pallas as pl


def kernel(x_rel0, x_rel1, edge_index_rel0, edge_index_rel1, W1_rel0, b1_rel0, W2_rel0, b2_rel0, W1_rel1, b1_rel1, W2_rel1, b2_rel1, W3, b3):
    raise NotImplementedError("write your pallas kernel here")


# speedup vs baseline: 2.4262x; 2.4262x over previous
import functools

import jax
import jax.numpy as jnp
from jax import lax
from jax.experimental import pallas as pl
from jax.experimental.pallas import tpu as pltpu
from jax.experimental.pallas import tpu_sc as plsc

N = 10000
D = 128
E = 160000
NC = 2
NS = 16
NP = 10240
RPT = NP // NS
EC = 128
EP = 163840
EPT = EP // NS
NCH = EPT // EC

_mesh = plsc.VectorSubcoreMesh(
    core_axis_name="c", subcore_axis_name="s", num_cores=NC, num_subcores=NS
)


def _leaky(x):
    return jnp.where(x >= 0, x, 0.01 * x)


@functools.partial(
    pl.kernel,
    out_type=jax.ShapeDtypeStruct((NC * NP, D), jnp.float32),
    mesh=_mesh,
    scratch_types=[
        pltpu.VMEM((EC,), jnp.int32),
        pltpu.VMEM((EC, D), jnp.float32),
        pltpu.VMEM((EC, D), jnp.float32),
        pltpu.VMEM_SHARED((NP, D), jnp.float32),
    ],
)
def _sc_degree(dst_hbm, out_hbm, idx_d, ones_v, zeros_v, acc):
    c = lax.axis_index("c")
    s = lax.axis_index("s")

    def fillo(i, _):
        for j in range(D // 16):
            ones_v[i, pl.ds(j * 16, 16)] = jnp.ones((16,), jnp.float32)
        return 0

    def fillz(i, _):
        for j in range(D // 16):
            zeros_v[i, pl.ds(j * 16, 16)] = jnp.zeros((16,), jnp.float32)
        return 0

    lax.fori_loop(0, EC, fillo, 0)
    lax.fori_loop(0, EC, fillz, 0)
    zero_base = s * RPT

    def zrow(i, _):
        pltpu.sync_copy(zeros_v, acc.at[pl.ds(zero_base + i * EC, EC)])
        return 0

    lax.fori_loop(0, RPT // EC, zrow, 0)
    plsc.subcore_barrier()

    base = c * EP + s * EPT

    def chunk(i, _):
        off = pl.multiple_of(base + i * EC, 8)
        pltpu.sync_copy(dst_hbm.at[pl.ds(off, EC)], idx_d)
        pltpu.sync_copy(ones_v, acc.at[idx_d], add=True)
        return 0

    lax.fori_loop(0, NCH, chunk, 0)
    plsc.subcore_barrier()
    row0 = s * RPT
    pltpu.sync_copy(acc.at[pl.ds(row0, RPT)], out_hbm.at[pl.ds(c * NP + row0, RPT)])


@functools.partial(
    pl.kernel,
    out_type=jax.ShapeDtypeStruct((NC * NP, D), jnp.float32),
    mesh=_mesh,
    scratch_types=[
        pltpu.VMEM((EC,), jnp.int32),
        pltpu.VMEM((EC,), jnp.int32),
        pltpu.VMEM((EC, D), jnp.float32),
        pltpu.VMEM_SHARED((NP, D), jnp.float32),
        pltpu.SemaphoreType.DMA,
    ],
)
def _sc_scatter_pass(g_hbm, src_hbm, dst_hbm, out_hbm, idx_s, idx_d, rows, acc, gsem):
    c = lax.axis_index("c")
    s = lax.axis_index("s")

    def fillz(i, _):
        for j in range(D // 16):
            rows[i, pl.ds(j * 16, 16)] = jnp.zeros((16,), jnp.float32)
        return 0

    lax.fori_loop(0, EC, fillz, 0)
    zero_base = s * RPT

    def zrow(i, _):
        pltpu.sync_copy(rows, acc.at[pl.ds(zero_base + i * EC, EC)])
        return 0

    lax.fori_loop(0, RPT // EC, zrow, 0)
    plsc.subcore_barrier()

    ebase = c * EP + s * EPT
    goff = c * NP

    def chunk(i, _):
        off = pl.multiple_of(ebase + i * EC, 8)
        pltpu.sync_copy(src_hbm.at[pl.ds(off, EC)], idx_s)
        for j in range(EC // 16):
            sl = pl.ds(j * 16, 16)
            idx_s[sl] = idx_s[sl] + goff
        pltpu.async_copy(g_hbm.at[idx_s], rows, gsem).wait()
        pltpu.sync_copy(dst_hbm.at[pl.ds(off, EC)], idx_d)
        pltpu.sync_copy(rows, acc.at[idx_d], add=True)
        return 0

    lax.fori_loop(0, NCH, chunk, 0)
    plsc.subcore_barrier()
    row0 = s * RPT
    pltpu.sync_copy(acc.at[pl.ds(row0, RPT)], out_hbm.at[pl.ds(c * NP + row0, RPT)])


BN = 1024
NB = NP // BN


def _mlp_body(x_ref, w1_ref, b1_ref, w2_ref, b2_ref, deg_ref, h_ref, g_ref, di_ref):
    for r in range(2):
        x = x_ref[r]
        h = _leaky(
            lax.dot_general(x, w1_ref[r], (((1,), (1,)), ((), ())),
                            preferred_element_type=jnp.float32) + b1_ref[r]
        )
        h = _leaky(
            lax.dot_general(h, w2_ref[r], (((1,), (1,)), ((), ())),
                            preferred_element_type=jnp.float32) + b2_ref[r]
        )
        dinvb = lax.rsqrt(jnp.maximum(deg_ref[r], 1.0))
        h_ref[r] = h
        g_ref[r] = h * dinvb
        di_ref[r] = dinvb


def _tc_mlp(xs, w1s, b1s, w2s, b2s, deg):
    spec_n = pl.BlockSpec((2, BN, D), lambda i: (0, i, 0))
    return pl.pallas_call(
        _mlp_body,
        grid=(NB,),
        in_specs=[
            spec_n,
            pl.BlockSpec((2, D, D), lambda i: (0, 0, 0)),
            pl.BlockSpec((2, 1, D), lambda i: (0, 0, 0)),
            pl.BlockSpec((2, D, D), lambda i: (0, 0, 0)),
            pl.BlockSpec((2, 1, D), lambda i: (0, 0, 0)),
            spec_n,
        ],
        out_specs=[spec_n, spec_n, spec_n],
        out_shape=[jax.ShapeDtypeStruct((2, NP, D), jnp.float32)] * 3,
    )(xs, w1s, b1s, w2s, b2s, deg)


def _update_body(f_ref, agg_ref, di_ref, f1_ref, g1_ref):
    for r in range(2):
        f1 = f_ref[r] - agg_ref[r] * di_ref[r]
        f1_ref[r] = f1
        g1_ref[r] = f1 * di_ref[r]


def _tc_update(f0, agg, dinv):
    spec_n = pl.BlockSpec((2, BN, D), lambda i: (0, i, 0))
    return pl.pallas_call(
        _update_body,
        grid=(NB,),
        in_specs=[spec_n, spec_n, spec_n],
        out_specs=[spec_n, spec_n],
        out_shape=[jax.ShapeDtypeStruct((2, NP, D), jnp.float32)] * 2,
    )(f0, agg, dinv)


def _final_body(f0_ref, f1_ref, agg_ref, di_ref, g_ref, b3_ref, o_ref):
    acc = jnp.broadcast_to(2.0 * b3_ref[0], (BN, D))
    for r in range(2):
        f1 = f1_ref[r]
        f2 = f1 - agg_ref[r] * di_ref[r]
        acc = acc + lax.dot_general(f0_ref[r], g_ref[0], (((1,), (1,)), ((), ())),
                                    preferred_element_type=jnp.float32)
        acc = acc + lax.dot_general(f1, g_ref[1], (((1,), (1,)), ((), ())),
                                    preferred_element_type=jnp.float32)
        acc = acc + lax.dot_general(f2, g_ref[2], (((1,), (1,)), ((), ())),
                                    preferred_element_type=jnp.float32)
    o_ref[...] = _leaky(acc)


def _tc_final(f0, f1, agg2, dinv, gs, b3):
    spec_n = pl.BlockSpec((2, BN, D), lambda i: (0, i, 0))
    return pl.pallas_call(
        _final_body,
        grid=(NB,),
        in_specs=[
            spec_n, spec_n, spec_n, spec_n,
            pl.BlockSpec((3, D, D), lambda i: (0, 0, 0)),
            pl.BlockSpec((1, D), lambda i: (0, 0)),
        ],
        out_specs=pl.BlockSpec((BN, D), lambda i: (i, 0)),
        out_shape=jax.ShapeDtypeStruct((NP, D), jnp.float32),
    )(f0, f1, agg2, dinv, gs, b3)


def kernel(x_rel0, x_rel1, edge_index_rel0, edge_index_rel1,
           W1_rel0, b1_rel0, W2_rel0, b2_rel0,
           W1_rel1, b1_rel1, W2_rel1, b2_rel1, W3, b3):
    f32 = jnp.float32
    xs = jnp.zeros((2, NP, D), f32).at[:, :N, :].set(jnp.stack([x_rel0, x_rel1]))
    epad_s = jnp.zeros((EP - E,), jnp.int32)
    epad_d = jnp.full((EP - E,), N, jnp.int32)
    src_flat = jnp.concatenate(
        [edge_index_rel0[0], epad_s, edge_index_rel1[0], epad_s])
    dst_flat = jnp.concatenate(
        [edge_index_rel0[1], epad_d, edge_index_rel1[1], epad_d])
    w1s = jnp.stack([W1_rel0, W1_rel1])
    b1s = jnp.stack([b1_rel0, b1_rel1])[:, None, :]
    w2s = jnp.stack([W2_rel0, W2_rel1])
    b2s = jnp.stack([b2_rel0, b2_rel1])[:, None, :]
    B0, B1, B2 = W3[:, :D], W3[:, D:2 * D], W3[:, 2 * D:]
    gs = jnp.stack([3.0 * B0, -3.0 * B0 + 3.0 * B1,
                    0.75 * B0 - 1.5 * B1 + 0.75 * B2])
    b3r = b3[None, :]

    deg = _sc_degree(dst_flat).reshape(2, NP, D)
    f0, g0, dinv = _tc_mlp(xs, w1s, b1s, w2s, b2s, deg)
    agg1 = _sc_scatter_pass(g0.reshape(2 * NP, D), src_flat, dst_flat)
    f1, g1 = _tc_update(f0, agg1.reshape(2, NP, D), dinv)
    agg2 = _sc_scatter_pass(g1.reshape(2 * NP, D), src_flat, dst_flat)
    out = _tc_final(f0, f1, agg2.reshape(2, NP, D), dinv, gs, b3r)
    return out[:N]


# speedup vs baseline: 3.2295x; 1.3311x over previous
import functools

import jax
import jax.numpy as jnp
from jax import lax
from jax.experimental import pallas as pl
from jax.experimental.pallas import tpu as pltpu
from jax.experimental.pallas import tpu_sc as plsc

N = 10000
D = 128
E = 160000
NC = 2
NS = 16
NP = 10240
RPT = NP // NS
EC = 128
EP = 163840
EPT = EP // NS
NCH = EPT // EC

_mesh = plsc.VectorSubcoreMesh(
    core_axis_name="c", subcore_axis_name="s", num_cores=NC, num_subcores=NS
)


def _leaky(x):
    return jnp.where(x >= 0, x, 0.01 * x)


@functools.partial(
    pl.kernel,
    out_type=jax.ShapeDtypeStruct((NC * NP, D), jnp.float32),
    mesh=_mesh,
    scratch_types=[
        pltpu.VMEM((EC,), jnp.int32),
        pltpu.VMEM((EC, D), jnp.float32),
        pltpu.VMEM((EC, D), jnp.float32),
        pltpu.VMEM_SHARED((NP, D), jnp.float32),
    ],
)
def _sc_degree(dst_hbm, out_hbm, idx_d, ones_v, zeros_v, acc):
    c = lax.axis_index("c")
    s = lax.axis_index("s")

    def fillo(i, _):
        for j in range(D // 16):
            ones_v[i, pl.ds(j * 16, 16)] = jnp.ones((16,), jnp.float32)
        return 0

    def fillz(i, _):
        for j in range(D // 16):
            zeros_v[i, pl.ds(j * 16, 16)] = jnp.zeros((16,), jnp.float32)
        return 0

    lax.fori_loop(0, EC, fillo, 0)
    lax.fori_loop(0, EC, fillz, 0)
    zero_base = s * RPT

    def zrow(i, _):
        pltpu.sync_copy(zeros_v, acc.at[pl.ds(zero_base + i * EC, EC)])
        return 0

    lax.fori_loop(0, RPT // EC, zrow, 0)
    plsc.subcore_barrier()

    base = c * EP + s * EPT

    def chunk(i, _):
        off = pl.multiple_of(base + i * EC, 8)
        pltpu.sync_copy(dst_hbm.at[pl.ds(off, EC)], idx_d)
        pltpu.sync_copy(ones_v, acc.at[idx_d], add=True)
        return 0

    lax.fori_loop(0, NCH, chunk, 0)
    plsc.subcore_barrier()
    row0 = s * RPT
    pltpu.sync_copy(acc.at[pl.ds(row0, RPT)], out_hbm.at[pl.ds(c * NP + row0, RPT)])


@functools.partial(
    pl.kernel,
    out_type=jax.ShapeDtypeStruct((NC * NP, D), jnp.float32),
    mesh=_mesh,
    scratch_types=[
        pltpu.VMEM((NCH, EC), jnp.int32),
        pltpu.VMEM((EC,), jnp.int32),
        pltpu.VMEM((EC,), jnp.int32),
        pltpu.VMEM((EC,), jnp.int32),
        pltpu.VMEM((EC,), jnp.int32),
        pltpu.VMEM((EC, D), jnp.float32),
        pltpu.VMEM((EC, D), jnp.float32),
        pltpu.VMEM_SHARED((NP, D), jnp.float32),
        pltpu.SemaphoreType.DMA,
        pltpu.SemaphoreType.DMA,
        pltpu.SemaphoreType.DMA,
        pltpu.SemaphoreType.DMA,
    ],
)
def _sc_scatter_pass(g_hbm, pk_hbm, out_hbm, pk_v, si0, si1, di0, di1,
                     r0, r1, acc, gs0, gs1, ss0, ss1):
    c = lax.axis_index("c")
    s = lax.axis_index("s")
    rows = [r0, r1]
    sidx = [si0, si1]
    didx = [di0, di1]
    gsems = [gs0, gs1]
    ssems = [ss0, ss1]

    pltpu.sync_copy(pk_hbm.at[c, s], pk_v)
    goff = c * NP

    def unpack(j, t):
        for k in range(EC // 16):
            sl = pl.ds(k * 16, 16)
            p = pk_v[j, sl]
            sidx[t][sl] = (p & 0xFFFF) + goff
            didx[t][sl] = jnp.right_shift(p, 16)

    def fillz(i, _):
        for j in range(D // 16):
            r0[i, pl.ds(j * 16, 16)] = jnp.zeros((16,), jnp.float32)
        return 0

    lax.fori_loop(0, EC, fillz, 0)
    zero_base = s * RPT

    def zrow(i, _):
        pltpu.sync_copy(r0, acc.at[pl.ds(zero_base + i * EC, EC)])
        return 0

    lax.fori_loop(0, RPT // EC, zrow, 0)
    plsc.subcore_barrier()

    def wait_gather(b):
        pltpu.make_async_copy(g_hbm.at[sidx[b]], rows[b], gsems[b]).wait()

    def wait_scatter(b):
        pltpu.make_async_copy(rows[b], acc.at[didx[b]], ssems[b]).wait()

    unpack(0, 0)
    pltpu.async_copy(g_hbm.at[sidx[0]], rows[0], gsems[0])

    def pair(q, _):
        for b in range(2):
            i = q * 2 + b
            nb = b ^ 1

            @pl.when(i + 1 < NCH)
            def _():
                @pl.when(i + 1 >= 2)
                def _():
                    wait_scatter(nb)

                unpack(i + 1, nb)
                pltpu.async_copy(g_hbm.at[sidx[nb]], rows[nb], gsems[nb])

            wait_gather(b)
            pltpu.async_copy(rows[b], acc.at[didx[b]], ssems[b], add=True)
        return 0

    lax.fori_loop(0, NCH // 2, pair, 0)
    wait_scatter(0)
    wait_scatter(1)

    plsc.subcore_barrier()
    row0 = s * RPT
    pltpu.sync_copy(acc.at[pl.ds(row0, RPT)], out_hbm.at[pl.ds(c * NP + row0, RPT)])


BN = 1024
NB = NP // BN


def _mlp_body(x_ref, w1_ref, b1_ref, w2_ref, b2_ref, deg_ref, h_ref, g_ref, di_ref):
    for r in range(2):
        x = x_ref[r]
        h = _leaky(
            lax.dot_general(x, w1_ref[r], (((1,), (1,)), ((), ())),
                            preferred_element_type=jnp.float32) + b1_ref[r]
        )
        h = _leaky(
            lax.dot_general(h, w2_ref[r], (((1,), (1,)), ((), ())),
                            preferred_element_type=jnp.float32) + b2_ref[r]
        )
        dinvb = lax.rsqrt(jnp.maximum(deg_ref[r], 1.0))
        h_ref[r] = h
        g_ref[r] = h * dinvb
        di_ref[r] = dinvb


def _tc_mlp(xs, w1s, b1s, w2s, b2s, deg):
    spec_n = pl.BlockSpec((2, BN, D), lambda i: (0, i, 0))
    return pl.pallas_call(
        _mlp_body,
        grid=(NB,),
        in_specs=[
            spec_n,
            pl.BlockSpec((2, D, D), lambda i: (0, 0, 0)),
            pl.BlockSpec((2, 1, D), lambda i: (0, 0, 0)),
            pl.BlockSpec((2, D, D), lambda i: (0, 0, 0)),
            pl.BlockSpec((2, 1, D), lambda i: (0, 0, 0)),
            spec_n,
        ],
        out_specs=[spec_n, spec_n, spec_n],
        out_shape=[jax.ShapeDtypeStruct((2, NP, D), jnp.float32)] * 3,
    )(xs, w1s, b1s, w2s, b2s, deg)


def _update_body(f_ref, agg_ref, di_ref, f1_ref, g1_ref):
    for r in range(2):
        f1 = f_ref[r] - agg_ref[r] * di_ref[r]
        f1_ref[r] = f1
        g1_ref[r] = f1 * di_ref[r]


def _tc_update(f0, agg, dinv):
    spec_n = pl.BlockSpec((2, BN, D), lambda i: (0, i, 0))
    return pl.pallas_call(
        _update_body,
        grid=(NB,),
        in_specs=[spec_n, spec_n, spec_n],
        out_specs=[spec_n, spec_n],
        out_shape=[jax.ShapeDtypeStruct((2, NP, D), jnp.float32)] * 2,
    )(f0, agg, dinv)


def _final_body(f0_ref, f1_ref, agg_ref, di_ref, g_ref, b3_ref, o_ref):
    acc = jnp.broadcast_to(2.0 * b3_ref[0], (BN, D))
    for r in range(2):
        f1 = f1_ref[r]
        f2 = f1 - agg_ref[r] * di_ref[r]
        acc = acc + lax.dot_general(f0_ref[r], g_ref[0], (((1,), (1,)), ((), ())),
                                    preferred_element_type=jnp.float32)
        acc = acc + lax.dot_general(f1, g_ref[1], (((1,), (1,)), ((), ())),
                                    preferred_element_type=jnp.float32)
        acc = acc + lax.dot_general(f2, g_ref[2], (((1,), (1,)), ((), ())),
                                    preferred_element_type=jnp.float32)
    o_ref[...] = _leaky(acc)


def _tc_final(f0, f1, agg2, dinv, gs, b3):
    spec_n = pl.BlockSpec((2, BN, D), lambda i: (0, i, 0))
    return pl.pallas_call(
        _final_body,
        grid=(NB,),
        in_specs=[
            spec_n, spec_n, spec_n, spec_n,
            pl.BlockSpec((3, D, D), lambda i: (0, 0, 0)),
            pl.BlockSpec((1, D), lambda i: (0, 0)),
        ],
        out_specs=pl.BlockSpec((BN, D), lambda i: (i, 0)),
        out_shape=jax.ShapeDtypeStruct((NP, D), jnp.float32),
    )(f0, f1, agg2, dinv, gs, b3)


def kernel(x_rel0, x_rel1, edge_index_rel0, edge_index_rel1,
           W1_rel0, b1_rel0, W2_rel0, b2_rel0,
           W1_rel1, b1_rel1, W2_rel1, b2_rel1, W3, b3):
    f32 = jnp.float32
    xs = jnp.zeros((2, NP, D), f32).at[:, :N, :].set(jnp.stack([x_rel0, x_rel1]))
    epad_s = jnp.zeros((EP - E,), jnp.int32)
    epad_d = jnp.full((EP - E,), N, jnp.int32)
    src_flat = jnp.concatenate(
        [edge_index_rel0[0], epad_s, edge_index_rel1[0], epad_s])
    dst_flat = jnp.concatenate(
        [edge_index_rel0[1], epad_d, edge_index_rel1[1], epad_d])
    pk4 = ((dst_flat << 16) | src_flat).reshape(NC, NS, NCH, EC)
    w1s = jnp.stack([W1_rel0, W1_rel1])
    b1s = jnp.stack([b1_rel0, b1_rel1])[:, None, :]
    w2s = jnp.stack([W2_rel0, W2_rel1])
    b2s = jnp.stack([b2_rel0, b2_rel1])[:, None, :]
    B0, B1, B2 = W3[:, :D], W3[:, D:2 * D], W3[:, 2 * D:]
    gs = jnp.stack([3.0 * B0, -3.0 * B0 + 3.0 * B1,
                    0.75 * B0 - 1.5 * B1 + 0.75 * B2])
    b3r = b3[None, :]

    deg = _sc_degree(dst_flat).reshape(2, NP, D)
    f0, g0, dinv = _tc_mlp(xs, w1s, b1s, w2s, b2s, deg)
    agg1 = _sc_scatter_pass(g0.reshape(2 * NP, D), pk4)
    f1, g1 = _tc_update(f0, agg1.reshape(2, NP, D), dinv)
    agg2 = _sc_scatter_pass(g1.reshape(2 * NP, D), pk4)
    out = _tc_final(f0, f1, agg2.reshape(2, NP, D), dinv, gs, b3r)
    return out[:N]
